# baseline (speedup 1.0000x reference)
import functools

import jax
import jax.numpy as jnp
from jax import lax
from jax.experimental import pallas as pl
from jax.experimental.pallas import tpu as pltpu
from jax.experimental.pallas import tpu_sc as plsc

NC = 2
NS = 16
NT = NC * NS
CHUNK = 64
ZROWS = 8
SHIFT = 14
MASK = (1 << SHIFT) - 1
NSEC = 2
NB = 2


def _sc_mesh():
    return plsc.VectorSubcoreMesh(core_axis_name="c", subcore_axis_name="s",
                                  num_cores=NC, num_subcores=NS)


def _deg_kernel(n_pad, n_chunks, d):
    rows_per_tile = n_pad // NS
    nz = rows_per_tile // ZROWS

    @functools.partial(
        pl.kernel,
        out_type=jax.ShapeDtypeStruct((NC, n_pad, d), jnp.float32),
        mesh=_sc_mesh(),
        scratch_types=[
            pltpu.VMEM((n_chunks, CHUNK), jnp.int32),
            pltpu.VMEM((CHUNK, d), jnp.float32),
            pltpu.VMEM((ZROWS, d), jnp.float32),
            pltpu.VMEM_SHARED((n_pad, d), jnp.float32),
        ],
    )
    def deg_kernel(pk_hbm, out_hbm, idx_v, ones_v, z_v, acc_sh):
        c = lax.axis_index("c")
        s = lax.axis_index("s")
        wid = c * NS + s
        pltpu.sync_copy(pk_hbm.at[wid], idx_v)

        def unpack(j, carry):
            for q in range(CHUNK // 16):
                v = idx_v[j, pl.ds(q * 16, 16)]
                idx_v[j, pl.ds(q * 16, 16)] = lax.shift_right_logical(v, SHIFT)
            return carry
        lax.fori_loop(0, n_chunks, unpack, 0)

        one = jnp.ones((16,), jnp.float32)
        zero = jnp.zeros((16,), jnp.float32)

        def fill_ones(r, carry):
            for q in range(d // 16):
                ones_v[r, pl.ds(q * 16, 16)] = one
            return carry
        lax.fori_loop(0, CHUNK, fill_ones, 0)

        def fill_zero(r, carry):
            for q in range(d // 16):
                z_v[r, pl.ds(q * 16, 16)] = zero
            return carry
        lax.fori_loop(0, ZROWS, fill_zero, 0)

        base = s * rows_per_tile

        def zero_acc(i, carry):
            pltpu.sync_copy(z_v, acc_sh.at[pl.ds(base + i * ZROWS, ZROWS)])
            return carry
        lax.fori_loop(0, nz, zero_acc, 0)

        plsc.subcore_barrier()

        def body(j, carry):
            pltpu.sync_copy(ones_v, acc_sh.at[idx_v.at[j]], add=True)
            return carry
        lax.fori_loop(0, n_chunks, body, 0)

        plsc.subcore_barrier()
        pltpu.sync_copy(acc_sh.at[pl.ds(base, rows_per_tile)],
                        out_hbm.at[c, pl.ds(base, rows_per_tile)])

    return deg_kernel


def _agg_kernel(n_pad, n_chunks, d):
    rows_per_tile = n_pad // NS
    nz = rows_per_tile // ZROWS
    sec_ch = n_chunks // NSEC

    @functools.partial(
        pl.kernel,
        out_type=jax.ShapeDtypeStruct((NC, n_pad, d), jnp.float32),
        mesh=_sc_mesh(),
        scratch_types=[
            pltpu.VMEM((sec_ch, CHUNK), jnp.int32),
            pltpu.VMEM((sec_ch, CHUNK), jnp.int32),
            [pltpu.VMEM((CHUNK, d), jnp.float32) for _ in range(NB)],
            pltpu.VMEM_SHARED((n_pad, d), jnp.float32),
            [pltpu.SemaphoreType.DMA for _ in range(NB)],
        ],
    )
    def agg_kernel(x2_hbm, pk_hbm, out_hbm,
                   src_v, dst_v, bufs, acc_sh, sems):
        c = lax.axis_index("c")
        s = lax.axis_index("s")
        wid = c * NS + s

        zero = jnp.zeros((16,), jnp.float32)

        def fill_zero(r, carry):
            for q in range(d // 16):
                bufs[0][r, pl.ds(q * 16, 16)] = zero
            return carry
        lax.fori_loop(0, ZROWS, fill_zero, 0)

        base = s * rows_per_tile

        def zero_acc(i, carry):
            pltpu.sync_copy(bufs[0].at[pl.ds(0, ZROWS)],
                            acc_sh.at[pl.ds(base + i * ZROWS, ZROWS)])
            return carry
        lax.fori_loop(0, nz, zero_acc, 0)

        plsc.subcore_barrier()

        def section(sec, carry):
            pltpu.sync_copy(pk_hbm.at[wid, pl.ds(sec * sec_ch, sec_ch)], src_v)

            def unpack(j, carry2):
                for q in range(CHUNK // 16):
                    v = src_v[j, pl.ds(q * 16, 16)]
                    dst_v[j, pl.ds(q * 16, 16)] = lax.shift_right_logical(
                        v, SHIFT)
                    src_v[j, pl.ds(q * 16, 16)] = lax.bitwise_and(v, MASK)
                return carry2
            lax.fori_loop(0, sec_ch, unpack, 0)

            for b in range(NB - 1):
                pltpu.async_copy(x2_hbm.at[src_v.at[b]], bufs[b], sems[b])

            def body(jj, carry2):
                for b in range(NB):
                    j = jj * NB + b
                    bn = (b + NB - 1) % NB
                    pltpu.async_copy(x2_hbm.at[src_v.at[j + NB - 1]],
                                     bufs[bn], sems[bn])
                    pltpu.make_async_copy(x2_hbm.at[src_v.at[j]], bufs[b],
                                          sems[b]).wait()
                    pltpu.sync_copy(bufs[b], acc_sh.at[dst_v.at[j]], add=True)
                return carry2
            lax.fori_loop(0, sec_ch // NB - 1, body, 0)

            for b in range(NB):
                j = sec_ch - NB + b
                if b == 0:
                    pltpu.async_copy(x2_hbm.at[src_v.at[sec_ch - 1]],
                                     bufs[NB - 1], sems[NB - 1])
                pltpu.make_async_copy(x2_hbm.at[src_v.at[j]], bufs[b],
                                      sems[b]).wait()
                pltpu.sync_copy(bufs[b], acc_sh.at[dst_v.at[j]], add=True)
            return carry
        lax.fori_loop(0, NSEC, section, 0)

        plsc.subcore_barrier()
        pltpu.sync_copy(acc_sh.at[pl.ds(base, rows_per_tile)],
                        out_hbm.at[c, pl.ds(base, rows_per_tile)])

    return agg_kernel


def _scale_body(deg_ref, x_ref, o_ref, dis_ref):
    dcol = deg_ref[0][:, :1] + deg_ref[1][:, :1]
    dis = jnp.where(dcol > 0, lax.rsqrt(jnp.where(dcol > 0, dcol, 1.0)), 0.0)
    o_ref[...] = x_ref[...] * dis
    dis_ref[...] = dis


def _finish_body(acc_ref, dis_ref, w_ref, b_ref, o_ref):
    agg = acc_ref[0] + acc_ref[1]
    h = jnp.dot(agg, w_ref[...], preferred_element_type=jnp.float32,
                precision=lax.Precision.HIGHEST)
    o_ref[...] = dis_ref[...] * h + b_ref[...]


def kernel(x, edge_index, W, b):
    n, d_in = x.shape
    d_out = W.shape[1]
    e = edge_index.shape[1]

    n_pad = ((n + 1 + NS * ZROWS - 1) // (NS * ZROWS)) * (NS * ZROWS)
    cm = CHUNK * NSEC * NB
    e_per_tile = ((e + NT * cm - 1) // (NT * cm)) * cm
    n_chunks = e_per_tile // CHUNK
    e_pad = e_per_tile * NT

    src = edge_index[0]
    dst = edge_index[1]
    pk = jnp.bitwise_or(jnp.left_shift(dst, SHIFT), src)
    et = -(-e // NT)
    if e % NT:
        pad0 = NT * et - e
        dums0 = n + (jnp.arange(pad0, dtype=jnp.int32) % (n_pad - n))
        pk = jnp.concatenate([pk, jnp.left_shift(dums0, SHIFT)])
    ppt = e_per_tile - et
    dums = n + (jnp.arange(ppt, dtype=jnp.int32) % (n_pad - n))
    pk_p = jnp.concatenate(
        [pk.reshape(NT, et),
         jnp.broadcast_to(jnp.left_shift(dums, SHIFT)[None, :], (NT, ppt))],
        axis=1,
    ).reshape(NT, n_chunks, CHUNK)

    deg_p = _deg_kernel(n_pad, n_chunks, d_in)(pk_p)

    blk = 1000 if n % 1000 == 0 else min(n, 1024)
    grid = (n + blk - 1) // blk

    x2, dis = pl.pallas_call(
        _scale_body,
        grid=(grid,),
        in_specs=[
            pl.BlockSpec((NC, blk, d_in), lambda i: (0, i, 0)),
            pl.BlockSpec((blk, d_in), lambda i: (i, 0)),
        ],
        out_specs=[
            pl.BlockSpec((blk, d_in), lambda i: (i, 0)),
            pl.BlockSpec((blk, 1), lambda i: (i, 0)),
        ],
        out_shape=[
            jax.ShapeDtypeStruct((n, d_in), jnp.float32),
            jax.ShapeDtypeStruct((n, 1), jnp.float32),
        ],
    )(deg_p, x)

    acc_p = _agg_kernel(n_pad, n_chunks, d_in)(x2, pk_p)

    out = pl.pallas_call(
        _finish_body,
        grid=(grid,),
        in_specs=[
            pl.BlockSpec((NC, blk, d_in), lambda i: (0, i, 0)),
            pl.BlockSpec((blk, 1), lambda i: (i, 0)),
            pl.BlockSpec((d_in, d_out), lambda i: (0, 0)),
            pl.BlockSpec((d_out,), lambda i: (0,)),
        ],
        out_specs=pl.BlockSpec((blk, d_out), lambda i: (i, 0)),
        out_shape=jax.ShapeDtypeStruct((n, d_out), jnp.float32),
    )(acc_p, dis, W, b)

    return out


# speedup vs baseline: 1.0062x; 1.0062x over previous
import functools

import jax
import jax.numpy as jnp
from jax import lax
from jax.experimental import pallas as pl
from jax.experimental.pallas import tpu as pltpu
from jax.experimental.pallas import tpu_sc as plsc

NC = 2
NS = 16
NT = NC * NS
CHUNK = 64
ZROWS = 8
SHIFT = 14
MASK = (1 << SHIFT) - 1
NSEC = 2
NB = 2


def _sc_mesh():
    return plsc.VectorSubcoreMesh(core_axis_name="c", subcore_axis_name="s",
                                  num_cores=NC, num_subcores=NS)


def _deg_kernel(n_pad, n_chunks, d, ck):
    rows_per_tile = n_pad // NS
    nz = rows_per_tile // ZROWS

    @functools.partial(
        pl.kernel,
        out_type=jax.ShapeDtypeStruct((NC, n_pad, d), jnp.float32),
        mesh=_sc_mesh(),
        scratch_types=[
            pltpu.VMEM((n_chunks, ck), jnp.int32),
            pltpu.VMEM((ck, d), jnp.float32),
            pltpu.VMEM((ZROWS, d), jnp.float32),
            pltpu.VMEM_SHARED((n_pad, d), jnp.float32),
        ],
    )
    def deg_kernel(pk_hbm, out_hbm, idx_v, ones_v, z_v, acc_sh):
        c = lax.axis_index("c")
        s = lax.axis_index("s")
        wid = c * NS + s
        pltpu.sync_copy(pk_hbm.at[wid], idx_v)

        def unpack(j, carry):
            for q in range(ck // 16):
                v = idx_v[j, pl.ds(q * 16, 16)]
                idx_v[j, pl.ds(q * 16, 16)] = lax.shift_right_logical(v, SHIFT)
            return carry
        lax.fori_loop(0, n_chunks, unpack, 0)

        one = jnp.ones((16,), jnp.float32)
        zero = jnp.zeros((16,), jnp.float32)

        def fill_ones(r, carry):
            for q in range(d // 16):
                ones_v[r, pl.ds(q * 16, 16)] = one
            return carry
        lax.fori_loop(0, ck, fill_ones, 0)

        def fill_zero(r, carry):
            for q in range(d // 16):
                z_v[r, pl.ds(q * 16, 16)] = zero
            return carry
        lax.fori_loop(0, ZROWS, fill_zero, 0)

        base = s * rows_per_tile

        def zero_acc(i, carry):
            pltpu.sync_copy(z_v, acc_sh.at[pl.ds(base + i * ZROWS, ZROWS)])
            return carry
        lax.fori_loop(0, nz, zero_acc, 0)

        plsc.subcore_barrier()

        def body(j, carry):
            pltpu.sync_copy(ones_v, acc_sh.at[idx_v.at[j]], add=True)
            return carry
        lax.fori_loop(0, n_chunks, body, 0)

        plsc.subcore_barrier()
        pltpu.sync_copy(acc_sh.at[pl.ds(base, rows_per_tile)],
                        out_hbm.at[c, pl.ds(base, rows_per_tile)])

    return deg_kernel


def _agg_kernel(n_pad, n_chunks, d):
    rows_per_tile = n_pad // NS
    nz = rows_per_tile // ZROWS
    sec_ch = n_chunks // NSEC

    @functools.partial(
        pl.kernel,
        out_type=jax.ShapeDtypeStruct((NC, n_pad, d), jnp.float32),
        mesh=_sc_mesh(),
        scratch_types=[
            pltpu.VMEM((sec_ch, CHUNK), jnp.int32),
            pltpu.VMEM((sec_ch, CHUNK), jnp.int32),
            [pltpu.VMEM((CHUNK, d), jnp.float32) for _ in range(NB)],
            pltpu.VMEM_SHARED((n_pad, d), jnp.float32),
            [pltpu.SemaphoreType.DMA for _ in range(NB)],
        ],
    )
    def agg_kernel(x2_hbm, pk_hbm, out_hbm,
                   src_v, dst_v, bufs, acc_sh, sems):
        c = lax.axis_index("c")
        s = lax.axis_index("s")
        wid = c * NS + s

        zero = jnp.zeros((16,), jnp.float32)

        def fill_zero(r, carry):
            for q in range(d // 16):
                bufs[0][r, pl.ds(q * 16, 16)] = zero
            return carry
        lax.fori_loop(0, ZROWS, fill_zero, 0)

        base = s * rows_per_tile

        def zero_acc(i, carry):
            pltpu.sync_copy(bufs[0].at[pl.ds(0, ZROWS)],
                            acc_sh.at[pl.ds(base + i * ZROWS, ZROWS)])
            return carry
        lax.fori_loop(0, nz, zero_acc, 0)

        plsc.subcore_barrier()

        def section(sec, carry):
            pltpu.sync_copy(pk_hbm.at[wid, pl.ds(sec * sec_ch, sec_ch)], src_v)

            def unpack(j, carry2):
                for q in range(CHUNK // 16):
                    v = src_v[j, pl.ds(q * 16, 16)]
                    dst_v[j, pl.ds(q * 16, 16)] = lax.shift_right_logical(
                        v, SHIFT)
                    src_v[j, pl.ds(q * 16, 16)] = lax.bitwise_and(v, MASK)
                return carry2
            lax.fori_loop(0, sec_ch, unpack, 0)

            for b in range(NB - 1):
                pltpu.async_copy(x2_hbm.at[src_v.at[b]], bufs[b], sems[b])

            def body(jj, carry2):
                for b in range(NB):
                    j = jj * NB + b
                    bn = (b + NB - 1) % NB
                    pltpu.async_copy(x2_hbm.at[src_v.at[j + NB - 1]],
                                     bufs[bn], sems[bn])
                    pltpu.make_async_copy(x2_hbm.at[src_v.at[j]], bufs[b],
                                          sems[b]).wait()
                    pltpu.sync_copy(bufs[b], acc_sh.at[dst_v.at[j]], add=True)
                return carry2
            lax.fori_loop(0, sec_ch // NB - 1, body, 0)

            for b in range(NB):
                j = sec_ch - NB + b
                if b == 0:
                    pltpu.async_copy(x2_hbm.at[src_v.at[sec_ch - 1]],
                                     bufs[NB - 1], sems[NB - 1])
                pltpu.make_async_copy(x2_hbm.at[src_v.at[j]], bufs[b],
                                      sems[b]).wait()
                pltpu.sync_copy(bufs[b], acc_sh.at[dst_v.at[j]], add=True)
            return carry
        lax.fori_loop(0, NSEC, section, 0)

        plsc.subcore_barrier()
        pltpu.sync_copy(acc_sh.at[pl.ds(base, rows_per_tile)],
                        out_hbm.at[c, pl.ds(base, rows_per_tile)])

    return agg_kernel


def _scale_body(deg_ref, x_ref, o_ref, dis_ref):
    dcol = deg_ref[0][:, :1] + deg_ref[1][:, :1]
    dis = jnp.where(dcol > 0, lax.rsqrt(jnp.where(dcol > 0, dcol, 1.0)), 0.0)
    o_ref[...] = x_ref[...] * dis
    dis_ref[...] = dis


def _finish_body(acc_ref, dis_ref, w_ref, b_ref, o_ref):
    agg = acc_ref[0] + acc_ref[1]
    h = jnp.dot(agg, w_ref[...], preferred_element_type=jnp.float32,
                precision=lax.Precision.HIGHEST)
    o_ref[...] = dis_ref[...] * h + b_ref[...]


def kernel(x, edge_index, W, b):
    n, d_in = x.shape
    d_out = W.shape[1]
    e = edge_index.shape[1]

    n_pad = ((n + 1 + NS * ZROWS - 1) // (NS * ZROWS)) * (NS * ZROWS)
    cm = CHUNK * NSEC * NB
    e_per_tile = ((e + NT * cm - 1) // (NT * cm)) * cm
    n_chunks = e_per_tile // CHUNK
    e_pad = e_per_tile * NT

    src = edge_index[0]
    dst = edge_index[1]
    pk = jnp.bitwise_or(jnp.left_shift(dst, SHIFT), src)
    et = -(-e // NT)
    if e % NT:
        pad0 = NT * et - e
        dums0 = n + (jnp.arange(pad0, dtype=jnp.int32) % (n_pad - n))
        pk = jnp.concatenate([pk, jnp.left_shift(dums0, SHIFT)])
    ppt = e_per_tile - et
    dums = n + (jnp.arange(ppt, dtype=jnp.int32) % (n_pad - n))
    pk_p = jnp.concatenate(
        [pk.reshape(NT, et),
         jnp.broadcast_to(jnp.left_shift(dums, SHIFT)[None, :], (NT, ppt))],
        axis=1,
    ).reshape(NT, n_chunks, CHUNK)

    deg_ck = 128
    deg_p = _deg_kernel(n_pad, e_per_tile // deg_ck, d_in, deg_ck)(
        pk_p.reshape(NT, e_per_tile // deg_ck, deg_ck))

    blk = 1000 if n % 1000 == 0 else min(n, 1024)
    grid = (n + blk - 1) // blk

    x2, dis = pl.pallas_call(
        _scale_body,
        grid=(grid,),
        in_specs=[
            pl.BlockSpec((NC, blk, d_in), lambda i: (0, i, 0)),
            pl.BlockSpec((blk, d_in), lambda i: (i, 0)),
        ],
        out_specs=[
            pl.BlockSpec((blk, d_in), lambda i: (i, 0)),
            pl.BlockSpec((blk, 1), lambda i: (i, 0)),
        ],
        out_shape=[
            jax.ShapeDtypeStruct((n, d_in), jnp.float32),
            jax.ShapeDtypeStruct((n, 1), jnp.float32),
        ],
    )(deg_p, x)

    acc_p = _agg_kernel(n_pad, n_chunks, d_in)(x2, pk_p)

    out = pl.pallas_call(
        _finish_body,
        grid=(grid,),
        in_specs=[
            pl.BlockSpec((NC, blk, d_in), lambda i: (0, i, 0)),
            pl.BlockSpec((blk, 1), lambda i: (i, 0)),
            pl.BlockSpec((d_in, d_out), lambda i: (0, 0)),
            pl.BlockSpec((d_out,), lambda i: (0,)),
        ],
        out_specs=pl.BlockSpec((blk, d_out), lambda i: (i, 0)),
        out_shape=jax.ShapeDtypeStruct((n, d_out), jnp.float32),
    )(acc_p, dis, W, b)

    return out
